# baseline (speedup 1.0000x reference)
import functools

import jax
import jax.numpy as jnp
from jax import lax
from jax.experimental import pallas as pl
from jax.experimental.pallas import tpu as pltpu
from jax.experimental.pallas import tpu_sc as plsc

_B, _N, _D, _K, _C = 64, 512, 128, 512, 11
_BB = 8
_R = _BB * _N

_NC, _NS, _L = 2, 16, 16
_NW = _NC * _NS
_BPW = _B // _NW


def _idx_rows(des, cneg, cnorm, kkf):
    dnorm = jnp.sum(des * des, axis=-1, keepdims=True)
    dot = lax.dot_general(des, cneg, (((1,), (1,)), ((), ())))
    d2 = (dnorm + dot) + cnorm
    m = jnp.min(d2, axis=-1, keepdims=True)
    idxf = jnp.min(jnp.where(d2 == m, kkf, float(_K)), axis=-1, keepdims=True)
    return idxf.astype(jnp.int32).reshape(_BB, _N)


def _tc1_body(da_ref, dg_ref, ca_ref, cg_ref, ia_ref, ig_ref,
              cneg_ref, cnorm_ref, kkf_ref):
    b = pl.program_id(0)

    @pl.when(b == 0)
    def _():
        ca = ca_ref[...]
        cg = cg_ref[...]
        cneg_ref[0] = -2.0 * ca
        cneg_ref[1] = -2.0 * cg
        cnorm_ref[0, :] = jnp.sum(ca * ca, axis=-1)
        cnorm_ref[1, :] = jnp.sum(cg * cg, axis=-1)
        kkf_ref[...] = lax.broadcasted_iota(
            jnp.int32, (1, _K), 1).astype(jnp.float32)

    kkf = kkf_ref[...]
    ia_ref[...] = _idx_rows(da_ref[...].reshape(_R, _D), cneg_ref[0],
                            cnorm_ref[0, :][None, :], kkf)
    ig_ref[...] = _idx_rows(dg_ref[...].reshape(_R, _D), cneg_ref[1],
                            cnorm_ref[1, :][None, :], kkf)


def _tc1(des_a, des_g, ca, cg):
    return pl.pallas_call(
        _tc1_body,
        grid=(_B // _BB,),
        in_specs=[
            pl.BlockSpec((_BB, _N, _D), lambda b: (b, 0, 0)),
            pl.BlockSpec((_BB, _N, _D), lambda b: (b, 0, 0)),
            pl.BlockSpec((_K, _D), lambda b: (0, 0)),
            pl.BlockSpec((_K, _D), lambda b: (0, 0)),
        ],
        out_specs=[
            pl.BlockSpec((_BB, _N), lambda b: (b, 0)),
            pl.BlockSpec((_BB, _N), lambda b: (b, 0)),
        ],
        out_shape=[
            jax.ShapeDtypeStruct((_B, _N), jnp.int32),
            jax.ShapeDtypeStruct((_B, _N), jnp.int32),
        ],
        scratch_shapes=[
            pltpu.VMEM((2, _K, _D), jnp.float32),
            pltpu.VMEM((2, _K), jnp.float32),
            pltpu.VMEM((1, _K), jnp.float32),
        ],
        compiler_params=pltpu.CompilerParams(
            dimension_semantics=("arbitrary",),
        ),
    )(des_a, des_g, ca, cg)


def _sc_hist(idxa, idxg):
    mesh = plsc.VectorSubcoreMesh(core_axis_name="c", subcore_axis_name="s")

    @functools.partial(
        pl.kernel, mesh=mesh,
        out_type=jax.ShapeDtypeStruct((_B, 2 * _K), jnp.float32),
        scratch_types=[
            pltpu.VMEM((_N,), jnp.int32),
            pltpu.VMEM((_L * _K,), jnp.float32),
            pltpu.VMEM((_K,), jnp.float32),
        ],
        compiler_params=pltpu.CompilerParams(needs_layout_passes=False),
    )
    def k(ia_hbm, ig_hbm, out_hbm, idx_v, hl_v, hist_v):
        wid = lax.axis_index("s") * _NC + lax.axis_index("c")
        lane = jnp.arange(_L, dtype=jnp.int32)
        base = lane * _K
        ones = jnp.full((_L,), 1.0 / _N, dtype=jnp.float32)

        for u in range(2 * _BPW):
            bb = wid * _BPW + (u % _BPW)
            src = ia_hbm if u < _BPW else ig_hbm
            col0 = 0 if u < _BPW else _K

            pltpu.sync_copy(src.at[bb], idx_v)

            def zero_body(i, _):
                hl_v[pl.ds(i * _L, _L)] = jnp.zeros((_L,), jnp.float32)
                return _
            lax.fori_loop(0, _L * _K // _L, zero_body, 0, unroll=8)

            for i in range(_N // _L):
                vec = idx_v[pl.ds(i * _L, _L)]
                plsc.addupdate_scatter(hl_v, [base + vec], ones)

            def red_body(j, _):
                acc = hl_v[pl.ds(j * _L, _L)]
                for l in range(1, _L):
                    acc = acc + hl_v[pl.ds(l * _K + j * _L, _L)]
                hist_v[pl.ds(j * _L, _L)] = acc
                return _
            lax.fori_loop(0, _K // _L, red_body, 0)

            pltpu.sync_copy(hist_v, out_hbm.at[bb, pl.ds(col0, _K)])

    return k(idxa, idxg)


def _tc2_body(hist_ref, w1_ref, b1_ref, w2_ref, b2_ref, out_ref):
    h = lax.dot_general(hist_ref[...], w1_ref[...], (((1,), (1,)), ((), ())))
    h = jnp.maximum(h + b1_ref[...][None, :], 0.0)
    logits = lax.dot_general(h, w2_ref[...], (((1,), (1,)), ((), ())))
    out_ref[...] = logits + b2_ref[...][None, :]


def _tc2(hist, W1, b1, W2, b2):
    return pl.pallas_call(
        _tc2_body,
        out_shape=jax.ShapeDtypeStruct((_B, _C), jnp.float32),
    )(hist, W1, b1, W2, b2)


def kernel(des_a, des_g, centroids_a, centroids_g, W1, b1, W2, b2):
    idxa, idxg = _tc1(des_a, des_g, centroids_a, centroids_g)
    hist = _sc_hist(idxa, idxg)
    return _tc2(hist, W1, b1, W2, b2)


# speedup vs baseline: 1.6534x; 1.6534x over previous
import jax
import jax.numpy as jnp
from jax import lax
from jax.experimental import pallas as pl
from jax.experimental.pallas import tpu as pltpu

_B, _N, _D, _K, _C = 64, 512, 128, 512, 11
_BB = 8
_R = _BB * _N


def _hist_rows(des, cneg, cnorm, out_sl):
    dnorm = jnp.sum(des * des, axis=-1, keepdims=True)
    dot = lax.dot_general(des, cneg, (((1,), (1,)), ((), ())))
    d2 = (dnorm + dot) + cnorm
    m = jnp.min(d2, axis=-1, keepdims=True)
    mask = (d2 == m).astype(jnp.float32)
    cnt = jnp.sum(mask.reshape(_BB, _N, _K), axis=1)
    out_sl[...] = cnt * (1.0 / _N)
    total = jnp.sum(cnt)

    @pl.when(total != float(_R))
    def _():
        kk = lax.broadcasted_iota(jnp.int32, (_R, _K), 1)
        idx = jnp.min(jnp.where(d2 == m, kk, _K), axis=-1, keepdims=True)
        onehot = (kk == idx).astype(jnp.float32)
        out_sl[...] = jnp.sum(onehot.reshape(_BB, _N, _K), axis=1) * (1.0 / _N)


def _body(da_ref, dg_ref, ca_ref, cg_ref, w1_ref, b1_ref, w2_ref, b2_ref,
          out_ref, hist_ref, cneg_ref, cnorm_ref):
    b = pl.program_id(0)

    @pl.when(b == 0)
    def _():
        ca = ca_ref[...]
        cg = cg_ref[...]
        cneg_ref[0] = -2.0 * ca
        cneg_ref[1] = -2.0 * cg
        cnorm_ref[0, :] = jnp.sum(ca * ca, axis=-1)
        cnorm_ref[1, :] = jnp.sum(cg * cg, axis=-1)

    row0 = pl.multiple_of(b * _BB, _BB)
    _hist_rows(da_ref[...].reshape(_R, _D), cneg_ref[0],
               cnorm_ref[0, :][None, :],
               hist_ref.at[pl.ds(row0, _BB), pl.ds(0, _K)])
    _hist_rows(dg_ref[...].reshape(_R, _D), cneg_ref[1],
               cnorm_ref[1, :][None, :],
               hist_ref.at[pl.ds(row0, _BB), pl.ds(_K, _K)])

    @pl.when(b == _B // _BB - 1)
    def _():
        hist = hist_ref[...]
        h = lax.dot_general(hist, w1_ref[...], (((1,), (1,)), ((), ())))
        h = jnp.maximum(h + b1_ref[...][None, :], 0.0)
        logits = lax.dot_general(h, w2_ref[...], (((1,), (1,)), ((), ())))
        out_ref[...] = logits + b2_ref[...][None, :]


def kernel(des_a, des_g, centroids_a, centroids_g, W1, b1, W2, b2):
    return pl.pallas_call(
        _body,
        grid=(_B // _BB,),
        in_specs=[
            pl.BlockSpec((_BB, _N, _D), lambda b: (b, 0, 0)),
            pl.BlockSpec((_BB, _N, _D), lambda b: (b, 0, 0)),
            pl.BlockSpec((_K, _D), lambda b: (0, 0)),
            pl.BlockSpec((_K, _D), lambda b: (0, 0)),
            pl.BlockSpec((_K, 2 * _K), lambda b: (0, 0)),
            pl.BlockSpec((_K,), lambda b: (0,)),
            pl.BlockSpec((_C, _K), lambda b: (0, 0)),
            pl.BlockSpec((_C,), lambda b: (0,)),
        ],
        out_specs=pl.BlockSpec((_B, _C), lambda b: (0, 0)),
        out_shape=jax.ShapeDtypeStruct((_B, _C), jnp.float32),
        scratch_shapes=[
            pltpu.VMEM((_B, 2 * _K), jnp.float32),
            pltpu.VMEM((2, _K, _D), jnp.float32),
            pltpu.VMEM((2, _K), jnp.float32),
        ],
        compiler_params=pltpu.CompilerParams(
            dimension_semantics=("arbitrary",),
        ),
    )(des_a, des_g, centroids_a, centroids_g, W1, b1, W2, b2)
